# Initial kernel scaffold; baseline (speedup 1.0000x reference)
import jax
import jax.numpy as jnp
from jax.experimental import pallas as pl


def kernel(partial_charges, idx_m, Rij, idx_i, idx_j, R, cell):
    raise NotImplementedError("write your pallas kernel here")


# speedup vs baseline: 45.3885x; 45.3885x over previous
import functools
import math

import jax
import jax.numpy as jnp
import numpy as np
from jax import lax
from jax.experimental import pallas as pl
from jax.experimental.pallas import tpu as pltpu
from jax.experimental.pallas import tpu_sc as plsc

KE = 14.3996
ALPHA = 0.3
K_MAX = 3
CUTOFF = 10.0
N_ATOMS = 50000
N_PAIRS = 1600000
N_MOL = 100
AT_PER_MOL = N_ATOMS // N_MOL

SQRT_ALPHA = math.sqrt(ALPHA)
F_CUT = math.erfc(SQRT_ALPHA * CUTOFF) / CUTOFF
SELF_C = math.sqrt(ALPHA / math.pi)

_P = 0.3275911
_A1 = 0.254829592
_A2 = -0.284496736
_A3 = 1.421413741
_A4 = -1.453152027
_A5 = 1.061405429


def _gen_kvecs_np():
    krange = np.arange(0, K_MAX + 1, dtype=np.float32)
    krange = np.concatenate([krange, -krange[1:]])
    a, b, c = np.meshgrid(krange, krange, krange, indexing="ij")
    kvecs = np.stack([a.ravel(), b.ravel(), c.ravel()], axis=1)
    norm = np.sum(kvecs**2, axis=1)
    mask = (norm <= K_MAX**2 + 2) & (norm != 0)
    return kvecs[mask].astype(np.float32)


_KVECS = _gen_kvecs_np()
NK = _KVECS.shape[0]

NC = 2
NS = 16
NW = NC * NS
LANES = 16
PAIRS_PER_W = N_PAIRS // NW
CHUNK = 10000
NCHUNK = PAIRS_PER_W // CHUNK
MOL_PAD = 128
BINS = LANES * MOL_PAD

PAIR_BLK = N_PAIRS // N_MOL


def _tc_main_body(rijt_ref, r_ref, q_ref, kv_ref, coef_ref, w_ref, yr_ref):
    x = rijt_ref[0, :]
    y = rijt_ref[1, :]
    z = rijt_ref[2, :]
    d2 = x * x + y * y + z * z
    d = jnp.sqrt(d2)
    a = SQRT_ALPHA * d
    t = 1.0 / (1.0 + _P * a)
    poly = t * (_A1 + t * (_A2 + t * (_A3 + t * (_A4 + t * _A5))))
    er = poly * jnp.exp(-a * a)
    fr = er / d
    w_ref[...] = jnp.where(d2 <= CUTOFF * CUTOFF, fr - F_CUT, 0.0)[None, None, :]

    rb = r_ref[0]
    qb = q_ref[0, 0]
    kvt = kv_ref[0]
    theta = (
        rb[:, 0:1] * kvt[0, :][None, :]
        + rb[:, 1:2] * kvt[1, :][None, :]
        + rb[:, 2:3] * kvt[2, :][None, :]
    )
    qc = qb[:, None]
    qr = jnp.sum(qc * jnp.cos(theta), axis=0)
    qi = jnp.sum(qc * jnp.sin(theta), axis=0)
    dens = qr * qr + qi * qi
    recip = jnp.sum(dens * coef_ref[0, 0])
    self_i = SELF_C * jnp.sum(qb * qb)
    yr_ref[...] = jnp.full((1, 1, 128), KE * (recip - self_i), jnp.float32)


_tc_main = pl.pallas_call(
    _tc_main_body,
    grid=(N_MOL,),
    in_specs=[
        pl.BlockSpec((3, PAIR_BLK), lambda m: (0, m)),
        pl.BlockSpec((1, AT_PER_MOL, 3), lambda m: (m, 0, 0)),
        pl.BlockSpec((1, 1, AT_PER_MOL), lambda m: (m, 0, 0)),
        pl.BlockSpec((1, 3, NK), lambda m: (m, 0, 0)),
        pl.BlockSpec((1, 1, NK), lambda m: (m, 0, 0)),
    ],
    out_specs=[
        pl.BlockSpec((1, 1, PAIR_BLK), lambda m: (m, 0, 0)),
        pl.BlockSpec((1, 1, 128), lambda m: (m, 0, 0)),
    ],
    out_shape=[
        jax.ShapeDtypeStruct((N_MOL, 1, PAIR_BLK), jnp.float32),
        jax.ShapeDtypeStruct((N_MOL, 1, 128), jnp.float32),
    ],
)


def _sc_pair_body(q_hbm, ii_hbm, jj_hbm, w_hbm, out_hbm, q_v, ii_v, jj_v, w_v, bins_v):
    wid = lax.axis_index("s") * NC + lax.axis_index("c")
    base = wid * PAIRS_PER_W
    pltpu.sync_copy(q_hbm, q_v)

    def zero_body(i, _):
        bins_v[pl.ds(i * LANES, LANES)] = jnp.zeros((LANES,), jnp.float32)
        return 0

    lax.fori_loop(0, BINS // LANES, zero_body, 0)

    laneb = lax.iota(jnp.int32, LANES) * MOL_PAD

    def chunk_body(ci, _):
        cbase = base + ci * CHUNK
        pltpu.sync_copy(ii_hbm.at[pl.ds(cbase, CHUNK)], ii_v)
        pltpu.sync_copy(jj_hbm.at[pl.ds(cbase, CHUNK)], jj_v)
        pltpu.sync_copy(w_hbm.at[pl.ds(cbase, CHUNK)], w_v)

        def pair_body(i, _):
            off = i * LANES
            ii = ii_v[pl.ds(off, LANES)]
            jj = jj_v[pl.ds(off, LANES)]
            wv = w_v[pl.ds(off, LANES)]
            qi = plsc.load_gather(q_v, [ii])
            qj = plsc.load_gather(q_v, [jj])
            mol = lax.div(ii, AT_PER_MOL)
            pot = qi * qj * wv
            plsc.addupdate_scatter(bins_v, [laneb + mol], pot)
            return 0

        lax.fori_loop(0, CHUNK // LANES, pair_body, 0)
        return 0

    lax.fori_loop(0, NCHUNK, chunk_body, 0)
    pltpu.sync_copy(bins_v, out_hbm.at[wid])


@functools.lru_cache(maxsize=None)
def _get_sc_pairs():
    return pl.kernel(
        _sc_pair_body,
        out_type=jax.ShapeDtypeStruct((NW, BINS), jnp.float32),
        mesh=plsc.VectorSubcoreMesh(
            core_axis_name="c", subcore_axis_name="s", num_cores=NC, num_subcores=NS
        ),
        compiler_params=pltpu.CompilerParams(needs_layout_passes=False),
        scratch_types=[
            pltpu.VMEM((N_ATOMS,), jnp.float32),
            pltpu.VMEM((CHUNK,), jnp.int32),
            pltpu.VMEM((CHUNK,), jnp.int32),
            pltpu.VMEM((CHUNK,), jnp.float32),
            pltpu.VMEM((BINS,), jnp.float32),
        ],
    )


def _tc_combine_body(part_ref, yrl_ref, out_ref):
    s = jnp.sum(part_ref[...], axis=0, keepdims=True)
    out_ref[...] = yrl_ref[...] + (0.5 * KE) * s


_tc_combine = pl.pallas_call(
    _tc_combine_body,
    out_shape=jax.ShapeDtypeStruct((1, 128), jnp.float32),
)


def kernel(partial_charges, idx_m, Rij, idx_i, idx_j, R, cell):
    q = partial_charges[:, 0]

    recip_box = 2.0 * np.pi * jnp.transpose(jnp.linalg.inv(cell), (0, 2, 1))
    v_box = jnp.abs(jnp.linalg.det(cell))
    prefactor = 2.0 * np.pi / v_box
    kv = jnp.matmul(jnp.asarray(_KVECS)[None, :, :], recip_box)
    k2 = jnp.sum(kv * kv, axis=2)
    coef = prefactor[:, None] * jnp.exp(-0.25 * k2 / ALPHA) / k2

    rijt = Rij.T
    r3 = R.reshape(N_MOL, AT_PER_MOL, 3)
    q2 = q.reshape(N_MOL, 1, AT_PER_MOL)
    coef = coef.reshape(N_MOL, 1, NK)

    kvt = jnp.swapaxes(kv, 1, 2)
    w, yr = _tc_main(rijt, r3, q2, kvt, coef)
    part = _get_sc_pairs()(q, idx_i, idx_j, w.reshape(N_PAIRS))
    part2 = part.reshape(NW * LANES, MOL_PAD)
    yrl = jnp.pad(yr[:, 0, 0], (0, 128 - N_MOL)).reshape(1, 128)
    outrow = _tc_combine(part2, yrl)
    return outrow[0, :N_MOL]
